# Initial kernel scaffold; baseline (speedup 1.0000x reference)
import jax
import jax.numpy as jnp
from jax.experimental import pallas as pl


def kernel(h, edge_index, ln1_scale, ln1_bias, W_self, W_neigh, b_conv, W_si, b_si, ln2_scale, ln2_bias):
    raise NotImplementedError("write your pallas kernel here")


# speedup vs baseline: 7.3414x; 7.3414x over previous
import functools

import jax
import jax.numpy as jnp
from jax import lax
from jax.experimental import pallas as pl
from jax.experimental.pallas import tpu as pltpu
from jax.experimental.pallas import tpu_sc as plsc

N = 10000
D = 128
TW = 144
E = 320000
NC = 2
NS = 16
NW = NC * NS
EW = E // NW
CH = 80
NK = EW // CH
NPAD = 10240
RPT = NPAD // NS

BN = 1000
NB = N // BN


def _elu(x):
    return jnp.where(x > 0, x, jnp.exp(jnp.minimum(x, 0.0)) - 1.0)


def _ln(x, scale, bias, eps=1e-5):
    mu = jnp.mean(x, axis=-1, keepdims=True)
    var = jnp.mean((x - mu) ** 2, axis=-1, keepdims=True)
    return (x - mu) / jnp.sqrt(var + eps) * scale + bias


def _ln1_body(h_ref, s_ref, b_ref, out_ref):
    y = _ln(h_ref[...], s_ref[...], b_ref[...])
    col = lax.broadcasted_iota(jnp.int32, (BN, TW - D), 1)
    extra = jnp.where(col == 0, 1.0, 0.0).astype(jnp.float32)
    out_ref[...] = jnp.concatenate([y, extra], axis=-1)


def _ln1(h, s, b):
    return pl.pallas_call(
        _ln1_body,
        grid=(NB,),
        in_specs=[
            pl.BlockSpec((BN, D), lambda i: (i, 0)),
            pl.BlockSpec((1, D), lambda i: (0, 0)),
            pl.BlockSpec((1, D), lambda i: (0, 0)),
        ],
        out_specs=pl.BlockSpec((BN, TW), lambda i: (i, 0)),
        out_shape=jax.ShapeDtypeStruct((N, TW), jnp.float32),
    )(h, s, b)


def _sc_body(src_hbm, dst_hbm, tab_hbm, z_hbm, out_hbm,
             src_v, dst_v, rows_v, acc_sh, sem):
    c = lax.axis_index("c")
    s = lax.axis_index("s")
    wid = s * NC + c

    pltpu.sync_copy(z_hbm, acc_sh.at[pl.ds(s * RPT, RPT)])
    pltpu.sync_copy(src_hbm.at[wid], src_v)
    pltpu.sync_copy(dst_hbm.at[wid], dst_v)
    plsc.subcore_barrier()

    @pl.loop(0, NK)
    def _chunk(k):
        pltpu.async_copy(tab_hbm.at[src_v.at[k]], rows_v, sem).wait()
        pltpu.sync_copy(rows_v, acc_sh.at[dst_v.at[k]], add=True)

    plsc.subcore_barrier()
    pltpu.sync_copy(acc_sh.at[pl.ds(s * RPT, RPT)],
                    out_hbm.at[c, pl.ds(s * RPT, RPT)])


@functools.cache
def _sc_agg_kernel():
    return pl.kernel(
        _sc_body,
        out_type=jax.ShapeDtypeStruct((NC, NPAD, TW), jnp.float32),
        mesh=plsc.VectorSubcoreMesh(core_axis_name="c", subcore_axis_name="s",
                                    num_cores=NC, num_subcores=NS),
        scratch_types=[
            pltpu.VMEM((NK, CH), jnp.int32),
            pltpu.VMEM((NK, CH), jnp.int32),
            pltpu.VMEM((CH, TW), jnp.float32),
            pltpu.VMEM_SHARED((NPAD, TW), jnp.float32),
            pltpu.SemaphoreType.DMA,
        ],
        compiler_params=pltpu.CompilerParams(use_tc_tiling_on_sc=False),
    )


def _post_body(hn_ref, agg_ref, ws_ref, wn_ref, bc_ref, wsi_ref, bsi_ref,
               s2_ref, b2_ref, out_ref):
    hn = hn_ref[:, :D]
    a = agg_ref[0] + agg_ref[1]
    deg = jnp.maximum(a[:, D:D + 1], 1.0)
    h_neigh = a[:, :D] / deg
    conv = (
        jnp.dot(hn, ws_ref[...], preferred_element_type=jnp.float32)
        + jnp.dot(h_neigh, wn_ref[...], preferred_element_type=jnp.float32)
        + bc_ref[...]
    )
    h1 = _elu(conv) + hn
    h2 = _ln(h1, s2_ref[...], b2_ref[...])
    h3 = _elu(
        jnp.dot(h2, wsi_ref[...], preferred_element_type=jnp.float32)
        + bsi_ref[...]
    )
    out_ref[...] = h3 + h2


def _post(hn_ext, agg, W_self, W_neigh, b_conv, W_si, b_si, s2, b2):
    full = lambda shape: pl.BlockSpec(shape, lambda i: tuple(0 for _ in shape))
    return pl.pallas_call(
        _post_body,
        grid=(NB,),
        in_specs=[
            pl.BlockSpec((BN, TW), lambda i: (i, 0)),
            pl.BlockSpec((NC, BN, TW), lambda i: (0, i, 0)),
            full((D, D)),
            full((D, D)),
            full((1, D)),
            full((D, D)),
            full((1, D)),
            full((1, D)),
            full((1, D)),
        ],
        out_specs=pl.BlockSpec((BN, D), lambda i: (i, 0)),
        out_shape=jax.ShapeDtypeStruct((N, D), jnp.float32),
    )(hn_ext, agg, W_self, W_neigh, b_conv, W_si, b_si, s2, b2)


def kernel(h, edge_index, ln1_scale, ln1_bias, W_self, W_neigh, b_conv,
           W_si, b_si, ln2_scale, ln2_bias):
    src = edge_index[0].astype(jnp.int32).reshape(NW, NK, CH)
    dst = edge_index[1].astype(jnp.int32).reshape(NW, NK, CH)
    zeros = jnp.zeros((RPT, TW), jnp.float32)

    hn_ext = _ln1(h, ln1_scale.reshape(1, D), ln1_bias.reshape(1, D))
    agg = _sc_agg_kernel()(src, dst, hn_ext, zeros)
    return _post(hn_ext, agg, W_self, W_neigh,
                 b_conv.reshape(1, D), W_si, b_si.reshape(1, D),
                 ln2_scale.reshape(1, D), ln2_bias.reshape(1, D))


# speedup vs baseline: 8.9206x; 1.2151x over previous
import functools

import jax
import jax.numpy as jnp
from jax import lax
from jax.experimental import pallas as pl
from jax.experimental.pallas import tpu as pltpu
from jax.experimental.pallas import tpu_sc as plsc

N = 10000
D = 128
TW = 144
E = 320000
NC = 2
NS = 16
NW = NC * NS
EW = E // NW
CH = 50
NK = EW // CH
R = 4
SG = 25
NSG = NK // SG
NPAD = 10240
RPT = NPAD // NS

BN = 1000
NB = N // BN


def _elu(x):
    return jnp.where(x > 0, x, jnp.exp(jnp.minimum(x, 0.0)) - 1.0)


def _ln(x, scale, bias, eps=1e-5):
    mu = jnp.mean(x, axis=-1, keepdims=True)
    var = jnp.mean((x - mu) ** 2, axis=-1, keepdims=True)
    return (x - mu) / jnp.sqrt(var + eps) * scale + bias


def _ln1_body(h_ref, s_ref, b_ref, out_ref):
    y = _ln(h_ref[...], s_ref[...], b_ref[...])
    col = lax.broadcasted_iota(jnp.int32, (BN, TW - D), 1)
    extra = jnp.where(col == 0, 1.0, 0.0).astype(jnp.float32)
    out_ref[...] = jnp.concatenate([y, extra], axis=-1)


def _ln1(h, s, b):
    return pl.pallas_call(
        _ln1_body,
        grid=(NB,),
        in_specs=[
            pl.BlockSpec((BN, D), lambda i: (i, 0)),
            pl.BlockSpec((1, D), lambda i: (0, 0)),
            pl.BlockSpec((1, D), lambda i: (0, 0)),
        ],
        out_specs=pl.BlockSpec((BN, TW), lambda i: (i, 0)),
        out_shape=jax.ShapeDtypeStruct((N, TW), jnp.float32),
    )(h, s, b)


def _sc_body(src_hbm, dst_hbm, tab_hbm, z_hbm, out_hbm,
             sidx, didx, rows_v, acc_sh, gsem, ssem):
    c = lax.axis_index("c")
    s = lax.axis_index("s")
    wid = s * NC + c

    pltpu.sync_copy(z_hbm, acc_sh.at[pl.ds(s * RPT, RPT)])
    plsc.subcore_barrier()

    def stage_idx(sg, slot):
        pltpu.sync_copy(src_hbm.at[wid, pl.ds(sg * SG, SG)], sidx.at[slot])
        pltpu.sync_copy(dst_hbm.at[wid, pl.ds(sg * SG, SG)], didx.at[slot])

    def fire_gather(k):
        slot2 = lax.rem(lax.div(k, SG), 2)
        pltpu.async_copy(tab_hbm.at[sidx.at[slot2, lax.rem(k, SG)]],
                         rows_v.at[lax.rem(k, R)], gsem.at[lax.rem(k, 2)])

    def wait_gather(k):
        pltpu.make_async_copy(tab_hbm.at[sidx.at[0, 0]], rows_v.at[0],
                              gsem.at[lax.rem(k, 2)]).wait()

    def fire_scatter(k):
        slot2 = lax.rem(lax.div(k, SG), 2)
        pltpu.async_copy(rows_v.at[lax.rem(k, R)],
                         acc_sh.at[didx.at[slot2, lax.rem(k, SG)]],
                         ssem.at[lax.rem(k, 2)], add=True)

    def wait_scatter(k):
        pltpu.make_async_copy(rows_v.at[0], acc_sh.at[didx.at[0, 0]],
                              ssem.at[lax.rem(k, 2)]).wait()

    stage_idx(0, 0)
    fire_gather(0)

    @pl.loop(0, NK)
    def _chunk(k):
        kn = k + 1

        @pl.when(kn >= R)
        def _free_slot():
            wait_scatter(kn - R)

        @pl.when(kn < NK)
        def _ahead():
            @pl.when(lax.rem(kn, SG) == 0)
            def _restage():
                stage_idx(lax.div(kn, SG), lax.rem(lax.div(kn, SG), 2))

            fire_gather(kn)

        wait_gather(k)
        fire_scatter(k)

    wait_scatter(NK - 3)
    wait_scatter(NK - 2)
    wait_scatter(NK - 1)
    plsc.subcore_barrier()
    pltpu.sync_copy(acc_sh.at[pl.ds(s * RPT, RPT)],
                    out_hbm.at[c, pl.ds(s * RPT, RPT)])


@functools.cache
def _sc_agg_kernel():
    return pl.kernel(
        _sc_body,
        out_type=jax.ShapeDtypeStruct((NC, NPAD, TW), jnp.float32),
        mesh=plsc.VectorSubcoreMesh(core_axis_name="c", subcore_axis_name="s",
                                    num_cores=NC, num_subcores=NS),
        scratch_types=[
            pltpu.VMEM((2, SG, CH), jnp.int32),
            pltpu.VMEM((2, SG, CH), jnp.int32),
            pltpu.VMEM((R, CH, TW), jnp.float32),
            pltpu.VMEM_SHARED((NPAD, TW), jnp.float32),
            pltpu.SemaphoreType.DMA((2,)),
            pltpu.SemaphoreType.DMA((2,)),
        ],
        compiler_params=pltpu.CompilerParams(use_tc_tiling_on_sc=False),
    )


def _post_body(hn_ref, agg_ref, ws_ref, wn_ref, bc_ref, wsi_ref, bsi_ref,
               s2_ref, b2_ref, out_ref):
    hn = hn_ref[:, :D]
    a = agg_ref[0] + agg_ref[1]
    deg = jnp.maximum(a[:, D:D + 1], 1.0)
    h_neigh = a[:, :D] / deg
    conv = (
        jnp.dot(hn, ws_ref[...], preferred_element_type=jnp.float32)
        + jnp.dot(h_neigh, wn_ref[...], preferred_element_type=jnp.float32)
        + bc_ref[...]
    )
    h1 = _elu(conv) + hn
    h2 = _ln(h1, s2_ref[...], b2_ref[...])
    h3 = _elu(
        jnp.dot(h2, wsi_ref[...], preferred_element_type=jnp.float32)
        + bsi_ref[...]
    )
    out_ref[...] = h3 + h2


def _post(hn_ext, agg, W_self, W_neigh, b_conv, W_si, b_si, s2, b2):
    full = lambda shape: pl.BlockSpec(shape, lambda i: tuple(0 for _ in shape))
    return pl.pallas_call(
        _post_body,
        grid=(NB,),
        in_specs=[
            pl.BlockSpec((BN, TW), lambda i: (i, 0)),
            pl.BlockSpec((NC, BN, TW), lambda i: (0, i, 0)),
            full((D, D)),
            full((D, D)),
            full((1, D)),
            full((D, D)),
            full((1, D)),
            full((1, D)),
            full((1, D)),
        ],
        out_specs=pl.BlockSpec((BN, D), lambda i: (i, 0)),
        out_shape=jax.ShapeDtypeStruct((N, D), jnp.float32),
    )(hn_ext, agg, W_self, W_neigh, b_conv, W_si, b_si, s2, b2)


def kernel(h, edge_index, ln1_scale, ln1_bias, W_self, W_neigh, b_conv,
           W_si, b_si, ln2_scale, ln2_bias):
    src = edge_index[0].astype(jnp.int32).reshape(NW, NK, CH)
    dst = edge_index[1].astype(jnp.int32).reshape(NW, NK, CH)
    zeros = jnp.zeros((RPT, TW), jnp.float32)

    hn_ext = _ln1(h, ln1_scale.reshape(1, D), ln1_bias.reshape(1, D))
    agg = _sc_agg_kernel()(src, dst, hn_ext, zeros)
    return _post(hn_ext, agg, W_self, W_neigh,
                 b_conv.reshape(1, D), W_si, b_si.reshape(1, D),
                 ln2_scale.reshape(1, D), ln2_bias.reshape(1, D))


# speedup vs baseline: 9.7514x; 1.0931x over previous
import functools

import jax
import jax.numpy as jnp
from jax import lax
from jax.experimental import pallas as pl
from jax.experimental.pallas import tpu as pltpu
from jax.experimental.pallas import tpu_sc as plsc

N = 10000
D = 128
TW = 144
E = 320000
NC = 2
NS = 16
NW = NC * NS
EW = E // NW
CH = 40
NK = EW // CH
R = 6
A = 2
SG = 25
NSG = NK // SG
NPAD = 10240
RPT = NPAD // NS

BN = 1000
NB = N // BN


def _elu(x):
    return jnp.where(x > 0, x, jnp.exp(jnp.minimum(x, 0.0)) - 1.0)


def _ln(x, scale, bias, eps=1e-5):
    mu = jnp.mean(x, axis=-1, keepdims=True)
    var = jnp.mean((x - mu) ** 2, axis=-1, keepdims=True)
    return (x - mu) / jnp.sqrt(var + eps) * scale + bias


def _ln1_body(h_ref, s_ref, b_ref, out_ref):
    y = _ln(h_ref[...], s_ref[...], b_ref[...])
    col = lax.broadcasted_iota(jnp.int32, (BN, TW - D), 1)
    extra = jnp.where(col == 0, 1.0, 0.0).astype(jnp.float32)
    out_ref[...] = jnp.concatenate([y, extra], axis=-1)


def _ln1(h, s, b):
    return pl.pallas_call(
        _ln1_body,
        grid=(NB,),
        in_specs=[
            pl.BlockSpec((BN, D), lambda i: (i, 0)),
            pl.BlockSpec((1, D), lambda i: (0, 0)),
            pl.BlockSpec((1, D), lambda i: (0, 0)),
        ],
        out_specs=pl.BlockSpec((BN, TW), lambda i: (i, 0)),
        out_shape=jax.ShapeDtypeStruct((N, TW), jnp.float32),
    )(h, s, b)


def _sc_body(src_hbm, dst_hbm, tab_hbm, z_hbm, out_hbm,
             sidx, didx, rows_v, acc_sh, gsem, ssem):
    c = lax.axis_index("c")
    s = lax.axis_index("s")
    wid = s * NC + c

    pltpu.sync_copy(z_hbm, acc_sh.at[pl.ds(s * RPT, RPT)])
    plsc.subcore_barrier()

    def stage_idx(sg, slot):
        pltpu.sync_copy(src_hbm.at[wid, pl.ds(sg * SG, SG)], sidx.at[slot])
        pltpu.sync_copy(dst_hbm.at[wid, pl.ds(sg * SG, SG)], didx.at[slot])

    def fire_gather(k):
        slot2 = lax.rem(lax.div(k, SG), 2)
        pltpu.async_copy(tab_hbm.at[sidx.at[slot2, lax.rem(k, SG)]],
                         rows_v.at[lax.rem(k, R)], gsem.at[lax.rem(k, 2)])

    def wait_gather(k):
        pltpu.make_async_copy(tab_hbm.at[sidx.at[0, 0]], rows_v.at[0],
                              gsem.at[lax.rem(k, 2)]).wait()

    def fire_scatter(k):
        slot2 = lax.rem(lax.div(k, SG), 2)
        pltpu.async_copy(rows_v.at[lax.rem(k, R)],
                         acc_sh.at[didx.at[slot2, lax.rem(k, SG)]],
                         ssem.at[lax.rem(k, 4)], add=True)

    def wait_scatter(k):
        pltpu.make_async_copy(rows_v.at[0], acc_sh.at[didx.at[0, 0]],
                              ssem.at[lax.rem(k, 4)]).wait()

    stage_idx(0, 0)
    for b in range(A):
        fire_gather(b)

    @pl.loop(0, NK)
    def _chunk(k):
        kn = k + A

        @pl.when(k >= R - A)
        def _free_slot():
            wait_scatter(k - (R - A))

        @pl.when(kn < NK)
        def _ahead():
            @pl.when(lax.rem(kn, SG) == 0)
            def _restage():
                stage_idx(lax.div(kn, SG), lax.rem(lax.div(kn, SG), 2))

            fire_gather(kn)

        wait_gather(k)
        fire_scatter(k)

    for t in range(NK - (R - A), NK):
        wait_scatter(t)
    plsc.subcore_barrier()
    pltpu.sync_copy(acc_sh.at[pl.ds(s * RPT, RPT)],
                    out_hbm.at[c, pl.ds(s * RPT, RPT)])


@functools.cache
def _sc_agg_kernel():
    return pl.kernel(
        _sc_body,
        out_type=jax.ShapeDtypeStruct((NC, NPAD, TW), jnp.float32),
        mesh=plsc.VectorSubcoreMesh(core_axis_name="c", subcore_axis_name="s",
                                    num_cores=NC, num_subcores=NS),
        scratch_types=[
            pltpu.VMEM((2, SG, CH), jnp.int32),
            pltpu.VMEM((2, SG, CH), jnp.int32),
            pltpu.VMEM((R, CH, TW), jnp.float32),
            pltpu.VMEM_SHARED((NPAD, TW), jnp.float32),
            pltpu.SemaphoreType.DMA((2,)),
            pltpu.SemaphoreType.DMA((4,)),
        ],
        compiler_params=pltpu.CompilerParams(use_tc_tiling_on_sc=False),
    )


def _post_body(hn_ref, agg_ref, ws_ref, wn_ref, bc_ref, wsi_ref, bsi_ref,
               s2_ref, b2_ref, out_ref):
    hn = hn_ref[:, :D]
    a = agg_ref[0] + agg_ref[1]
    deg = jnp.maximum(a[:, D:D + 1], 1.0)
    h_neigh = a[:, :D] / deg
    conv = (
        jnp.dot(hn, ws_ref[...], preferred_element_type=jnp.float32)
        + jnp.dot(h_neigh, wn_ref[...], preferred_element_type=jnp.float32)
        + bc_ref[...]
    )
    h1 = _elu(conv) + hn
    h2 = _ln(h1, s2_ref[...], b2_ref[...])
    h3 = _elu(
        jnp.dot(h2, wsi_ref[...], preferred_element_type=jnp.float32)
        + bsi_ref[...]
    )
    out_ref[...] = h3 + h2


def _post(hn_ext, agg, W_self, W_neigh, b_conv, W_si, b_si, s2, b2):
    full = lambda shape: pl.BlockSpec(shape, lambda i: tuple(0 for _ in shape))
    return pl.pallas_call(
        _post_body,
        grid=(NB,),
        in_specs=[
            pl.BlockSpec((BN, TW), lambda i: (i, 0)),
            pl.BlockSpec((NC, BN, TW), lambda i: (0, i, 0)),
            full((D, D)),
            full((D, D)),
            full((1, D)),
            full((D, D)),
            full((1, D)),
            full((1, D)),
            full((1, D)),
        ],
        out_specs=pl.BlockSpec((BN, D), lambda i: (i, 0)),
        out_shape=jax.ShapeDtypeStruct((N, D), jnp.float32),
    )(hn_ext, agg, W_self, W_neigh, b_conv, W_si, b_si, s2, b2)


def kernel(h, edge_index, ln1_scale, ln1_bias, W_self, W_neigh, b_conv,
           W_si, b_si, ln2_scale, ln2_bias):
    src = edge_index[0].astype(jnp.int32).reshape(NW, NK, CH)
    dst = edge_index[1].astype(jnp.int32).reshape(NW, NK, CH)
    zeros = jnp.zeros((RPT, TW), jnp.float32)

    hn_ext = _ln1(h, ln1_scale.reshape(1, D), ln1_bias.reshape(1, D))
    agg = _sc_agg_kernel()(src, dst, hn_ext, zeros)
    return _post(hn_ext, agg, W_self, W_neigh,
                 b_conv.reshape(1, D), W_si, b_si.reshape(1, D),
                 ln2_scale.reshape(1, D), ln2_bias.reshape(1, D))


# speedup vs baseline: 11.3273x; 1.1616x over previous
import functools

import jax
import jax.numpy as jnp
from jax import lax
from jax.experimental import pallas as pl
from jax.experimental.pallas import tpu as pltpu
from jax.experimental.pallas import tpu_sc as plsc

N = 10000
D = 128
TW = 144
E = 320000
NC = 2
NS = 16
NW = NC * NS
EW = E // NW
CH = 40
NK = EW // CH
R = 6
A = 3
SG = 25
NSG = NK // SG
NPAD = 10240
RPT = NPAD // NS

BN = 1000
NB = N // BN


def _elu(x):
    return jnp.where(x > 0, x, jnp.exp(jnp.minimum(x, 0.0)) - 1.0)


def _ln(x, scale, bias, eps=1e-5):
    mu = jnp.mean(x, axis=-1, keepdims=True)
    var = jnp.mean((x - mu) ** 2, axis=-1, keepdims=True)
    return (x - mu) / jnp.sqrt(var + eps) * scale + bias


def _ln1_body(h_ref, s_ref, b_ref, out_ref):
    y = _ln(h_ref[...], s_ref[...], b_ref[...])
    col = lax.broadcasted_iota(jnp.int32, (BN, TW - D), 1)
    extra = jnp.where(col == 0, 1.0, 0.0).astype(jnp.float32)
    out_ref[...] = jnp.concatenate([y, extra], axis=-1)


def _ln1(h, s, b):
    return pl.pallas_call(
        _ln1_body,
        grid=(NB,),
        in_specs=[
            pl.BlockSpec((BN, D), lambda i: (i, 0)),
            pl.BlockSpec((1, D), lambda i: (0, 0)),
            pl.BlockSpec((1, D), lambda i: (0, 0)),
        ],
        out_specs=pl.BlockSpec((BN, TW), lambda i: (i, 0)),
        out_shape=jax.ShapeDtypeStruct((N, TW), jnp.float32),
    )(h, s, b)


def _sc_body(src_hbm, dst_hbm, tab_hbm, out_hbm,
             sidx, didx, rows_v, acc_sh, gsem, ssem):
    c = lax.axis_index("c")
    s = lax.axis_index("s")
    wid = s * NC + c

    z16 = jnp.zeros((16,), jnp.float32)

    @pl.loop(0, CH)
    def _zrow(r):
        for c2 in range(TW // 16):
            rows_v[0, r, pl.ds(c2 * 16, 16)] = z16

    for i in range(RPT // CH):
        pltpu.async_copy(rows_v.at[0],
                         acc_sh.at[pl.ds(s * RPT + i * CH, CH)],
                         gsem.at[0])
    for i in range(RPT // CH):
        pltpu.make_async_copy(rows_v.at[0],
                              acc_sh.at[pl.ds(s * RPT, CH)],
                              gsem.at[0]).wait()
    plsc.subcore_barrier()

    def stage_idx(sg, slot):
        pltpu.sync_copy(src_hbm.at[wid, pl.ds(sg * SG, SG)], sidx.at[slot])
        pltpu.sync_copy(dst_hbm.at[wid, pl.ds(sg * SG, SG)], didx.at[slot])

    def fire_gather(k):
        slot2 = lax.rem(lax.div(k, SG), 2)
        pltpu.async_copy(tab_hbm.at[sidx.at[slot2, lax.rem(k, SG)]],
                         rows_v.at[lax.rem(k, R)], gsem.at[lax.rem(k, 4)])

    def wait_gather(k):
        pltpu.make_async_copy(tab_hbm.at[sidx.at[0, 0]], rows_v.at[0],
                              gsem.at[lax.rem(k, 4)]).wait()

    def fire_scatter(k):
        slot2 = lax.rem(lax.div(k, SG), 2)
        pltpu.async_copy(rows_v.at[lax.rem(k, R)],
                         acc_sh.at[didx.at[slot2, lax.rem(k, SG)]],
                         ssem.at[lax.rem(k, 4)], add=True)

    def wait_scatter(k):
        pltpu.make_async_copy(rows_v.at[0], acc_sh.at[didx.at[0, 0]],
                              ssem.at[lax.rem(k, 4)]).wait()

    stage_idx(0, 0)
    for b in range(A):
        fire_gather(b)

    @pl.loop(0, NK)
    def _chunk(k):
        kn = k + A

        @pl.when(k >= R - A)
        def _free_slot():
            wait_scatter(k - (R - A))

        @pl.when(kn < NK)
        def _ahead():
            @pl.when(lax.rem(kn, SG) == 0)
            def _restage():
                stage_idx(lax.div(kn, SG), lax.rem(lax.div(kn, SG), 2))

            fire_gather(kn)

        wait_gather(k)
        fire_scatter(k)

    for t in range(NK - (R - A), NK):
        wait_scatter(t)
    plsc.subcore_barrier()
    pltpu.sync_copy(acc_sh.at[pl.ds(s * RPT, RPT)],
                    out_hbm.at[c, pl.ds(s * RPT, RPT)])


@functools.cache
def _sc_agg_kernel():
    return pl.kernel(
        _sc_body,
        out_type=jax.ShapeDtypeStruct((NC, NPAD, TW), jnp.float32),
        mesh=plsc.VectorSubcoreMesh(core_axis_name="c", subcore_axis_name="s",
                                    num_cores=NC, num_subcores=NS),
        scratch_types=[
            pltpu.VMEM((2, SG, CH), jnp.int32),
            pltpu.VMEM((2, SG, CH), jnp.int32),
            pltpu.VMEM((R, CH, TW), jnp.float32),
            pltpu.VMEM_SHARED((NPAD, TW), jnp.float32),
            pltpu.SemaphoreType.DMA((4,)),
            pltpu.SemaphoreType.DMA((4,)),
        ],
        compiler_params=pltpu.CompilerParams(use_tc_tiling_on_sc=False),
    )


def _post_body(hn_ref, agg_ref, ws_ref, wn_ref, bc_ref, wsi_ref, bsi_ref,
               s2_ref, b2_ref, out_ref):
    hn = hn_ref[:, :D]
    a = agg_ref[0] + agg_ref[1]
    deg = jnp.maximum(a[:, D:D + 1], 1.0)
    h_neigh = a[:, :D] / deg
    conv = (
        jnp.dot(hn, ws_ref[...], preferred_element_type=jnp.float32)
        + jnp.dot(h_neigh, wn_ref[...], preferred_element_type=jnp.float32)
        + bc_ref[...]
    )
    h1 = _elu(conv) + hn
    h2 = _ln(h1, s2_ref[...], b2_ref[...])
    h3 = _elu(
        jnp.dot(h2, wsi_ref[...], preferred_element_type=jnp.float32)
        + bsi_ref[...]
    )
    out_ref[...] = h3 + h2


def _post(hn_ext, agg, W_self, W_neigh, b_conv, W_si, b_si, s2, b2):
    full = lambda shape: pl.BlockSpec(shape, lambda i: tuple(0 for _ in shape))
    return pl.pallas_call(
        _post_body,
        grid=(NB,),
        in_specs=[
            pl.BlockSpec((BN, TW), lambda i: (i, 0)),
            pl.BlockSpec((NC, BN, TW), lambda i: (0, i, 0)),
            full((D, D)),
            full((D, D)),
            full((1, D)),
            full((D, D)),
            full((1, D)),
            full((1, D)),
            full((1, D)),
        ],
        out_specs=pl.BlockSpec((BN, D), lambda i: (i, 0)),
        out_shape=jax.ShapeDtypeStruct((N, D), jnp.float32),
    )(hn_ext, agg, W_self, W_neigh, b_conv, W_si, b_si, s2, b2)


def kernel(h, edge_index, ln1_scale, ln1_bias, W_self, W_neigh, b_conv,
           W_si, b_si, ln2_scale, ln2_bias):
    src = edge_index[0].astype(jnp.int32).reshape(NW, NK, CH)
    dst = edge_index[1].astype(jnp.int32).reshape(NW, NK, CH)

    hn_ext = _ln1(h, ln1_scale.reshape(1, D), ln1_bias.reshape(1, D))
    agg = _sc_agg_kernel()(src, dst, hn_ext)
    return _post(hn_ext, agg, W_self, W_neigh,
                 b_conv.reshape(1, D), W_si, b_si.reshape(1, D),
                 ln2_scale.reshape(1, D), ln2_bias.reshape(1, D))


# speedup vs baseline: 11.3482x; 1.0019x over previous
import functools

import jax
import jax.numpy as jnp
from jax import lax
from jax.experimental import pallas as pl
from jax.experimental.pallas import tpu as pltpu
from jax.experimental.pallas import tpu_sc as plsc

N = 10000
D = 128
TW = 144
E = 320000
NC = 2
NS = 16
NW = NC * NS
EW = E // NW
CH = 40
NK = EW // CH
R = 6
A = 3
SG = 25
NSG = NK // SG
NPAD = 10240
RPT = NPAD // NS

BN = 1000
NB = N // BN


def _elu(x):
    return jnp.where(x > 0, x, jnp.exp(jnp.minimum(x, 0.0)) - 1.0)


def _ln(x, scale, bias, eps=1e-5):
    mu = jnp.mean(x, axis=-1, keepdims=True)
    var = jnp.mean((x - mu) ** 2, axis=-1, keepdims=True)
    return (x - mu) / jnp.sqrt(var + eps) * scale + bias


def _ln1_body(h_ref, s_ref, b_ref, out_ref):
    y = _ln(h_ref[...], s_ref[...], b_ref[...])
    col = lax.broadcasted_iota(jnp.int32, (BN, TW - D), 1)
    extra = jnp.where(col == 0, 1.0, 0.0).astype(jnp.float32)
    out_ref[...] = jnp.concatenate([y, extra], axis=-1)


def _ln1(h, s, b):
    return pl.pallas_call(
        _ln1_body,
        grid=(NB,),
        in_specs=[
            pl.BlockSpec((BN, D), lambda i: (i, 0)),
            pl.BlockSpec((1, D), lambda i: (0, 0)),
            pl.BlockSpec((1, D), lambda i: (0, 0)),
        ],
        out_specs=pl.BlockSpec((BN, TW), lambda i: (i, 0)),
        out_shape=jax.ShapeDtypeStruct((N, TW), jnp.float32),
    )(h, s, b)


def _sc_body(src_hbm, dst_hbm, tab_hbm, out_hbm,
             sidx, didx, rows_v, acc_sh, gsem, ssem):
    c = lax.axis_index("c")
    s = lax.axis_index("s")
    wid = s * NC + c

    z16 = jnp.zeros((16,), jnp.float32)

    @pl.loop(0, CH)
    def _zrow(r):
        for c2 in range(TW // 16):
            rows_v[0, r, pl.ds(c2 * 16, 16)] = z16

    for i in range(RPT // CH):
        pltpu.async_copy(rows_v.at[0],
                         acc_sh.at[pl.ds(s * RPT + i * CH, CH)],
                         gsem.at[0])
    for i in range(RPT // CH):
        pltpu.make_async_copy(rows_v.at[0],
                              acc_sh.at[pl.ds(s * RPT, CH)],
                              gsem.at[0]).wait()
    plsc.subcore_barrier()

    def stage_idx(sg, slot):
        pltpu.sync_copy(src_hbm.at[wid, pl.ds(sg * SG, SG)], sidx.at[slot])
        pltpu.sync_copy(dst_hbm.at[wid, pl.ds(sg * SG, SG)], didx.at[slot])

    def fire_gather(k):
        slot2 = lax.rem(lax.div(k, SG), 2)
        pltpu.async_copy(tab_hbm.at[sidx.at[slot2, lax.rem(k, SG)]],
                         rows_v.at[lax.rem(k, R)], gsem.at[lax.rem(k, 4)])

    def wait_gather(k):
        pltpu.make_async_copy(tab_hbm.at[sidx.at[0, 0]], rows_v.at[0],
                              gsem.at[lax.rem(k, 4)]).wait()

    def fire_scatter(k):
        slot2 = lax.rem(lax.div(k, SG), 2)
        pltpu.async_copy(rows_v.at[lax.rem(k, R)],
                         acc_sh.at[didx.at[slot2, lax.rem(k, SG)]],
                         ssem.at[lax.rem(k, 4)], add=True)

    def wait_scatter(k):
        pltpu.make_async_copy(rows_v.at[0], acc_sh.at[didx.at[0, 0]],
                              ssem.at[lax.rem(k, 4)]).wait()

    stage_idx(0, 0)
    for b in range(A):
        fire_gather(b)

    @pl.loop(0, NK)
    def _chunk(k):
        kn = k + A

        @pl.when(k >= R - A)
        def _free_slot():
            wait_scatter(k - (R - A))

        @pl.when(kn < NK)
        def _ahead():
            @pl.when(lax.rem(kn, SG) == 0)
            def _restage():
                stage_idx(lax.div(kn, SG), lax.rem(lax.div(kn, SG), 2))

            fire_gather(kn)

        wait_gather(k)
        fire_scatter(k)

    for t in range(NK - (R - A), NK):
        wait_scatter(t)
    plsc.subcore_barrier()
    pltpu.sync_copy(acc_sh.at[pl.ds(s * RPT, RPT)],
                    out_hbm.at[c, pl.ds(s * RPT, RPT)])


@functools.cache
def _sc_agg_kernel():
    return pl.kernel(
        _sc_body,
        out_type=jax.ShapeDtypeStruct((NC, NPAD, TW), jnp.float32),
        mesh=plsc.VectorSubcoreMesh(core_axis_name="c", subcore_axis_name="s",
                                    num_cores=NC, num_subcores=NS),
        scratch_types=[
            pltpu.VMEM((2, SG, CH), jnp.int32),
            pltpu.VMEM((2, SG, CH), jnp.int32),
            pltpu.VMEM((R, CH, TW), jnp.float32),
            pltpu.VMEM_SHARED((NPAD, TW), jnp.float32),
            pltpu.SemaphoreType.DMA((4,)),
            pltpu.SemaphoreType.DMA((4,)),
        ],
        compiler_params=pltpu.CompilerParams(use_tc_tiling_on_sc=False),
    )


def _post_body(hn_ref, agg_ref, ws_ref, wn_ref, bc_ref, wsi_ref, bsi_ref,
               s2_ref, b2_ref, out_ref):
    hn = hn_ref[...]
    a = agg_ref[0] + agg_ref[1]
    deg = jnp.maximum(a[:, D:D + 1], 1.0)
    h_neigh = a[:, :D] / deg
    conv = (
        jnp.dot(hn, ws_ref[...], preferred_element_type=jnp.float32)
        + jnp.dot(h_neigh, wn_ref[...], preferred_element_type=jnp.float32)
        + bc_ref[...]
    )
    h1 = _elu(conv) + hn
    h2 = _ln(h1, s2_ref[...], b2_ref[...])
    h3 = _elu(
        jnp.dot(h2, wsi_ref[...], preferred_element_type=jnp.float32)
        + bsi_ref[...]
    )
    out_ref[...] = h3 + h2


def _post(hn_ext, agg, W_self, W_neigh, b_conv, W_si, b_si, s2, b2):
    full = lambda shape: pl.BlockSpec(shape, lambda i: tuple(0 for _ in shape))
    return pl.pallas_call(
        _post_body,
        grid=(NB,),
        in_specs=[
            pl.BlockSpec((BN, D), lambda i: (i, 0)),
            pl.BlockSpec((NC, BN, TW), lambda i: (0, i, 0)),
            full((D, D)),
            full((D, D)),
            full((1, D)),
            full((D, D)),
            full((1, D)),
            full((1, D)),
            full((1, D)),
        ],
        out_specs=pl.BlockSpec((BN, D), lambda i: (i, 0)),
        out_shape=jax.ShapeDtypeStruct((N, D), jnp.float32),
    )(hn_ext, agg, W_self, W_neigh, b_conv, W_si, b_si, s2, b2)


def kernel(h, edge_index, ln1_scale, ln1_bias, W_self, W_neigh, b_conv,
           W_si, b_si, ln2_scale, ln2_bias):
    src = edge_index[0].astype(jnp.int32).reshape(NW, NK, CH)
    dst = edge_index[1].astype(jnp.int32).reshape(NW, NK, CH)

    hn_ext = _ln1(h, ln1_scale.reshape(1, D), ln1_bias.reshape(1, D))
    agg = _sc_agg_kernel()(src, dst, hn_ext)
    return _post(hn_ext, agg, W_self, W_neigh,
                 b_conv.reshape(1, D), W_si, b_si.reshape(1, D),
                 ln2_scale.reshape(1, D), ln2_bias.reshape(1, D))


# speedup vs baseline: 13.5547x; 1.1944x over previous
import functools

import jax
import jax.numpy as jnp
from jax import lax
from jax.experimental import pallas as pl
from jax.experimental.pallas import tpu as pltpu
from jax.experimental.pallas import tpu_sc as plsc

N = 10000
D = 128
E = 320000
NC = 2
NS = 16
NW = NC * NS
EW = E // NW
CH = 40
NK = EW // CH
R = 6
A = 3
SG = 25
NSG = NK // SG
NV = (SG * CH + 15) // 16
NPAD = 10240
RPT = NPAD // NS

BN = 1000
NB = N // BN


def _elu(x):
    return jnp.where(x > 0, x, jnp.exp(jnp.minimum(x, 0.0)) - 1.0)


def _ln(x, scale, bias, eps=1e-5):
    mu = jnp.mean(x, axis=-1, keepdims=True)
    var = jnp.mean((x - mu) ** 2, axis=-1, keepdims=True)
    return (x - mu) / jnp.sqrt(var + eps) * scale + bias


def _ln1_body(h_ref, s_ref, b_ref, out_ref):
    out_ref[...] = _ln(h_ref[...], s_ref[...], b_ref[...])


def _ln1(h, s, b):
    return pl.pallas_call(
        _ln1_body,
        grid=(NB,),
        in_specs=[
            pl.BlockSpec((BN, D), lambda i: (i, 0)),
            pl.BlockSpec((1, D), lambda i: (0, 0)),
            pl.BlockSpec((1, D), lambda i: (0, 0)),
        ],
        out_specs=pl.BlockSpec((BN, D), lambda i: (i, 0)),
        out_shape=jax.ShapeDtypeStruct((N, D), jnp.float32),
    )(h, s, b)


def _sc_body(src_hbm, dst_hbm, tab_hbm, out_hbm, deg_hbm,
             sidx, didx, rows_v, ones_v, zd_v, acc_sh, deg_sh,
             gsem, ssem, dsem):
    c = lax.axis_index("c")
    s = lax.axis_index("s")
    wid = s * NC + c

    z16 = jnp.zeros((16,), jnp.float32)
    one16 = jnp.ones((16,), jnp.float32)

    for r in range(3):
        ones_v[pl.ds(r * 16, 16)] = one16
    @pl.loop(0, RPT // 16)
    def _zd(i):
        zd_v[pl.ds(i * 16, 16)] = z16

    @pl.loop(0, CH)
    def _zrow(r):
        for c2 in range(D // 16):
            rows_v[0, r, pl.ds(c2 * 16, 16)] = z16

    for i in range(RPT // CH):
        pltpu.async_copy(rows_v.at[0],
                         acc_sh.at[pl.ds(s * RPT + i * CH, CH)],
                         gsem.at[0])
    pltpu.async_copy(zd_v, deg_sh.at[pl.ds(s * RPT, RPT)], gsem.at[0])
    for i in range(RPT // CH):
        pltpu.make_async_copy(rows_v.at[0],
                              acc_sh.at[pl.ds(s * RPT, CH)],
                              gsem.at[0]).wait()
    pltpu.make_async_copy(zd_v, deg_sh.at[pl.ds(s * RPT, RPT)],
                          gsem.at[0]).wait()
    plsc.subcore_barrier()

    def stage_idx(sg, slot):
        pltpu.sync_copy(src_hbm.at[wid, pl.ds(sg * SG, SG)], sidx.at[slot])
        pltpu.sync_copy(dst_hbm.at[wid, pl.ds(sg * SG, SG)], didx.at[slot])

    def fire_gather(k):
        slot2 = lax.rem(lax.div(k, SG), 2)
        pltpu.async_copy(tab_hbm.at[sidx.at[slot2, lax.rem(k, SG)]],
                         rows_v.at[lax.rem(k, R)], gsem.at[lax.rem(k, 4)])

    def wait_gather(k):
        pltpu.make_async_copy(tab_hbm.at[sidx.at[0, 0]], rows_v.at[0],
                              gsem.at[lax.rem(k, 4)]).wait()

    def fire_scatter(k):
        slot2 = lax.rem(lax.div(k, SG), 2)
        pltpu.async_copy(rows_v.at[lax.rem(k, R)],
                         acc_sh.at[didx.at[slot2, lax.rem(k, SG)]],
                         ssem.at[lax.rem(k, 4)], add=True)

    def wait_scatter(k):
        pltpu.make_async_copy(rows_v.at[0], acc_sh.at[didx.at[0, 0]],
                              ssem.at[lax.rem(k, 4)]).wait()

    def fire_deg(k):
        slot2 = lax.rem(lax.div(k, SG), 2)
        pltpu.async_copy(ones_v.at[pl.ds(0, CH)],
                         deg_sh.at[didx.at[slot2, lax.rem(k, SG)]],
                         dsem.at[lax.rem(k, 4)], add=True)

    def wait_deg(k):
        pltpu.make_async_copy(ones_v.at[pl.ds(0, CH)],
                              deg_sh.at[didx.at[0, 0]],
                              dsem.at[lax.rem(k, 4)]).wait()

    stage_idx(0, 0)
    for b in range(A):
        fire_gather(b)

    @pl.loop(0, NK)
    def _chunk(k):
        kn = k + A

        @pl.when(k >= R - A)
        def _free_slot():
            wait_scatter(k - (R - A))
            wait_deg(k - (R - A))

        @pl.when(kn < NK)
        def _ahead():
            @pl.when(lax.rem(kn, SG) == 0)
            def _restage():
                sg = lax.div(kn, SG)
                slot = lax.rem(sg, 2)
                stage_idx(sg, slot)

            fire_gather(kn)

        wait_gather(k)
        fire_scatter(k)
        fire_deg(k)

    for t in range(NK - (R - A), NK):
        wait_scatter(t)
        wait_deg(t)
    plsc.subcore_barrier()

    @pl.when(s == 0)
    def _wb_deg():
        for b in range(NB):
            pltpu.sync_copy(deg_sh.at[pl.ds(b * BN, BN)], deg_hbm.at[b, c])
    pltpu.sync_copy(acc_sh.at[pl.ds(s * RPT, RPT)],
                    out_hbm.at[c, pl.ds(s * RPT, RPT)])


@functools.cache
def _sc_agg_kernel():
    return pl.kernel(
        _sc_body,
        out_type=(
            jax.ShapeDtypeStruct((NC, NPAD, D), jnp.float32),
            jax.ShapeDtypeStruct((NB, NC, BN), jnp.float32),
        ),
        mesh=plsc.VectorSubcoreMesh(core_axis_name="c", subcore_axis_name="s",
                                    num_cores=NC, num_subcores=NS),
        scratch_types=[
            pltpu.VMEM((2, SG, CH), jnp.int32),
            pltpu.VMEM((2, SG, CH), jnp.int32),
            pltpu.VMEM((R, CH, D), jnp.float32),
            pltpu.VMEM((48,), jnp.float32),
            pltpu.VMEM((RPT,), jnp.float32),
            pltpu.VMEM_SHARED((NPAD, D), jnp.float32),
            pltpu.VMEM_SHARED((NPAD,), jnp.float32),
            pltpu.SemaphoreType.DMA((4,)),
            pltpu.SemaphoreType.DMA((4,)),
            pltpu.SemaphoreType.DMA((4,)),
        ],
        compiler_params=pltpu.CompilerParams(use_tc_tiling_on_sc=False),
    )


def _post_body(hn_ref, agg_ref, deg_ref, ws_ref, wn_ref, bc_ref, wsi_ref,
               bsi_ref, s2_ref, b2_ref, out_ref):
    hn = hn_ref[...]
    a = agg_ref[0] + agg_ref[1]
    dd = deg_ref[0]
    deg = jnp.maximum(jnp.transpose(dd[0:1] + dd[1:2]), 1.0)
    h_neigh = a / deg
    conv = (
        jnp.dot(hn, ws_ref[...], preferred_element_type=jnp.float32)
        + jnp.dot(h_neigh, wn_ref[...], preferred_element_type=jnp.float32)
        + bc_ref[...]
    )
    h1 = _elu(conv) + hn
    h2 = _ln(h1, s2_ref[...], b2_ref[...])
    h3 = _elu(
        jnp.dot(h2, wsi_ref[...], preferred_element_type=jnp.float32)
        + bsi_ref[...]
    )
    out_ref[...] = h3 + h2


def _post(hn, agg, deg, W_self, W_neigh, b_conv, W_si, b_si, s2, b2):
    full = lambda shape: pl.BlockSpec(shape, lambda i: tuple(0 for _ in shape))
    return pl.pallas_call(
        _post_body,
        grid=(NB,),
        in_specs=[
            pl.BlockSpec((BN, D), lambda i: (i, 0)),
            pl.BlockSpec((NC, BN, D), lambda i: (0, i, 0)),
            pl.BlockSpec((1, NC, BN), lambda i: (i, 0, 0)),
            full((D, D)),
            full((D, D)),
            full((1, D)),
            full((D, D)),
            full((1, D)),
            full((1, D)),
            full((1, D)),
        ],
        out_specs=pl.BlockSpec((BN, D), lambda i: (i, 0)),
        out_shape=jax.ShapeDtypeStruct((N, D), jnp.float32),
    )(hn, agg, deg, W_self, W_neigh, b_conv, W_si, b_si, s2, b2)


def kernel(h, edge_index, ln1_scale, ln1_bias, W_self, W_neigh, b_conv,
           W_si, b_si, ln2_scale, ln2_bias):
    src = edge_index[0].astype(jnp.int32).reshape(NW, NK, CH)
    dst = edge_index[1].astype(jnp.int32).reshape(NW, NK, CH)

    hn = _ln1(h, ln1_scale.reshape(1, D), ln1_bias.reshape(1, D))
    agg, deg = _sc_agg_kernel()(src, dst, hn)
    return _post(hn, agg, deg, W_self, W_neigh,
                 b_conv.reshape(1, D), W_si, b_si.reshape(1, D),
                 ln2_scale.reshape(1, D), ln2_bias.reshape(1, D))
